# Initial kernel scaffold; baseline (speedup 1.0000x reference)
import jax
import jax.numpy as jnp
from jax.experimental import pallas as pl


def kernel(obs_t, traj, eW1, eb1, eg1, ebt1, eW2, eb2, eg2, ebt2, eW3, eb3, dW1, db1, dg1, dbt1, dW2, db2, dg2, dbt2, dW3, db3, emb):
    raise NotImplementedError("write your pallas kernel here")


# speedup vs baseline: 1.1917x; 1.1917x over previous
import functools

import jax
import jax.numpy as jnp
from jax.experimental import pallas as pl

_ERF_P = (7.85386146e-05, -0.000801019371, 0.00518832775, -0.0268538129,
          0.112835854, -0.37612626, 1.12837911)
_ERFC_P = (0.0232682, -0.138703942, 0.368742466, -0.582473278, 0.621000462,
           -0.494451523, 0.340488, -0.274112701, 0.563825965)
_ERFC_R = (-10.477664, 12.9772, -7.49551868, 2.92101908, -1.01526523,
           0.42184633, -0.282076746, 0.564189494)


def _horner(y, coefs):
    acc = y * jnp.float32(coefs[0])
    for c in coefs[1:-1]:
        acc = (acc + jnp.float32(c)) * y
    return acc + jnp.float32(coefs[-1])


def _erfc(x):
    ax = jnp.abs(x)
    x2 = x * x
    one_minus_erf = 1.0 - x * _horner(x2, _ERF_P)
    nx2 = -x2
    z = jnp.exp(nx2)
    zq = z * (1.0 / ax)
    y = 1.0 / x2
    p = jnp.where(ax < 2.0, _horner(y, _ERFC_P), _horner(y, _ERFC_R))
    val = zq * p
    val = jnp.where(nx2 < -88.7228394, 0.0, val)
    val = jnp.where(x < 0.0, 2.0 - val, val)
    return jnp.where(ax < 1.0, one_minus_erf, val)


def _gelu(x):
    return (0.5 * x) * _erfc((-x) * 0.7071067690849304)


def _gelu_fast(x):
    return (0.5 * x) * (1.0 + jax.lax.erf(x * 0.7071067690849304))


def _tree_sum(x):
    w = x.shape[-1]
    acc = x[:, 0:128]
    for k in range(1, w // 128):
        acc = acc + x[:, 128 * k:128 * (k + 1)]
    a8 = acc[:, 0:8]
    for k in range(1, 16):
        a8 = a8 + acc[:, 8 * k:8 * (k + 1)]
    a4 = a8[:, 0:4] + a8[:, 4:8]
    a2 = a4[:, 0:2] + a4[:, 2:4]
    return a2[:, 0:1] + a2[:, 1:2]


def _ln_exact(x, g, b):
    w = x.shape[-1]
    m = _tree_sum(x) / jnp.float32(w)
    c = x - m
    v = _tree_sum(c * c) / jnp.float32(w)
    return (x - m) / jnp.sqrt(v + 1e-5) * g + b


def _ln_fast(x, g, b):
    m = jnp.mean(x, axis=-1, keepdims=True)
    v = jnp.var(x, axis=-1, keepdims=True)
    return (x - m) / jnp.sqrt(v + 1e-5) * g + b


def _vqvae_body(obs_ref, traj_ref,
                eW1_ref, eb1_ref, eg1_ref, ebt1_ref,
                eW2_ref, eb2_ref, eg2_ref, ebt2_ref,
                eW3_ref, eb3_ref,
                dW1_ref, db1_ref, dg1_ref, dbt1_ref,
                dW2_ref, db2_ref, dg2_ref, dbt2_ref,
                dW3_ref, db3_ref, emb_ref, embsq_ref,
                out_ref, zq_ref, ze_ref, idx_ref):
    obs = obs_ref[...]
    traj = traj_ref[...]

    x = jnp.concatenate([obs, traj], axis=1)
    h = _gelu(jnp.dot(x, eW1_ref[...]) + eb1_ref[...])
    h = _ln_exact(h, eg1_ref[...], ebt1_ref[...])
    h = _gelu(jnp.dot(h, eW2_ref[...]) + eb2_ref[...])
    h = _ln_exact(h, eg2_ref[...], ebt2_ref[...])
    ze = _gelu(jnp.dot(h, eW3_ref[...]) + eb3_ref[...])
    ze_ref[...] = ze

    emb = emb_ref[...]
    mm = jax.lax.dot_general(ze, emb, (((1,), (1,)), ((), ())))
    dist = (_tree_sum(ze * ze) - 2.0 * mm) + embsq_ref[...]
    rowmin = jnp.min(dist, axis=1, keepdims=True)
    iota = jax.lax.broadcasted_iota(jnp.int32, dist.shape, 1)
    idx = jnp.min(jnp.where(dist == rowmin, iota, jnp.int32(2 ** 30)), axis=1)
    idx_ref[...] = idx

    onehot = (iota == idx[:, None]).astype(jnp.float32)
    zq = jnp.dot(onehot, emb, precision=jax.lax.Precision.HIGHEST)
    zq_ref[...] = zq

    y = jnp.concatenate([obs, zq], axis=1)
    y = _gelu_fast(jnp.dot(y, dW1_ref[...]) + db1_ref[...])
    y = _ln_fast(y, dg1_ref[...], dbt1_ref[...])
    y = _gelu_fast(jnp.dot(y, dW2_ref[...]) + db2_ref[...])
    y = _ln_fast(y, dg2_ref[...], dbt2_ref[...])
    out_ref[...] = jnp.dot(y, dW3_ref[...]) + db3_ref[...]


@functools.partial(jax.jit, static_argnames=("interpret",))
def kernel(obs_t, traj, eW1, eb1, eg1, ebt1, eW2, eb2, eg2, ebt2, eW3, eb3,
           dW1, db1, dg1, dbt1, dW2, db2, dg2, dbt2, dW3, db3, emb,
           interpret=False):
    B = obs_t.shape[0]
    blk = 512
    grid = (B // blk,)

    embsq = jnp.sum(emb ** 2, axis=1)[None, :]

    def const_spec(a):
        return pl.BlockSpec(a.shape, lambda i: (0,) * a.ndim)

    in_specs = [
        pl.BlockSpec((blk, obs_t.shape[1]), lambda i: (i, 0)),
        pl.BlockSpec((blk, traj.shape[1]), lambda i: (i, 0)),
    ] + [const_spec(a) for a in (eW1, eb1, eg1, ebt1, eW2, eb2, eg2, ebt2,
                                 eW3, eb3, dW1, db1, dg1, dbt1, dW2, db2,
                                 dg2, dbt2, dW3, db3, emb, embsq)]

    out_shape = (
        jax.ShapeDtypeStruct((B, 768), jnp.float32),
        jax.ShapeDtypeStruct((B, 256), jnp.float32),
        jax.ShapeDtypeStruct((B, 256), jnp.float32),
        jax.ShapeDtypeStruct((B,), jnp.int32),
    )
    out_specs = (
        pl.BlockSpec((blk, 768), lambda i: (i, 0)),
        pl.BlockSpec((blk, 256), lambda i: (i, 0)),
        pl.BlockSpec((blk, 256), lambda i: (i, 0)),
        pl.BlockSpec((blk,), lambda i: (i,)),
    )

    out, zq, ze, idx = pl.pallas_call(
        _vqvae_body,
        grid=grid,
        in_specs=in_specs,
        out_specs=out_specs,
        out_shape=out_shape,
        interpret=interpret,
    )(obs_t, traj, eW1, eb1, eg1, ebt1, eW2, eb2, eg2, ebt2, eW3, eb3,
      dW1, db1, dg1, dbt1, dW2, db2, dg2, dbt2, dW3, db3, emb, embsq)
    return (out, zq, ze, idx)
